# baseline (speedup 1.0000x reference)
import functools
import math

import jax
import jax.numpy as jnp
from jax import lax
from jax.experimental import pallas as pl
from jax.experimental.pallas import tpu as pltpu
from jax.experimental.pallas import tpu_sc as plsc

D_MODEL = 64
D_PAD = 128
SCALE = math.sqrt(D_MODEL)
LANES = 16

NUM_CORES = 2
NUM_SUBCORES = 16
NUM_WORKERS = NUM_CORES * NUM_SUBCORES

CHUNK = 128


@jax.jit
def _embed_sc(x3d, w2):
    nw, n_chunks, _ = x3d.shape
    n_total = nw * n_chunks * CHUNK

    mesh = plsc.VectorSubcoreMesh(core_axis_name="c", subcore_axis_name="s")

    @functools.partial(
        pl.kernel,
        out_type=jax.ShapeDtypeStruct((n_total, D_PAD), jnp.float32),
        mesh=mesh,
        scratch_types=[
            pltpu.VMEM((n_chunks, CHUNK), jnp.int32),
            pltpu.VMEM((n_chunks, CHUNK), jnp.int32),
            pltpu.VMEM((CHUNK, D_PAD), jnp.float32),
            pltpu.VMEM((CHUNK, D_PAD), jnp.float32),
            pltpu.SemaphoreType.DMA,
            pltpu.SemaphoreType.DMA,
        ],
        compiler_params=pltpu.CompilerParams(use_tc_tiling_on_sc=False),
    )
    def body(w_hbm, idx_hbm, out_hbm, idx_v, idxh_v, rows_v, rows_o, gsem, osem):
        wid = lax.axis_index("s") * NUM_CORES + lax.axis_index("c")
        base = wid * n_chunks * CHUNK
        pltpu.sync_copy(idx_hbm.at[wid], idx_v)

        @pl.loop(0, n_chunks * CHUNK // LANES)
        def halve_loop(q):
            o = pl.multiple_of(q * LANES, LANES)
            r = o // CHUNK
            c = o % CHUNK
            idxh_v[r, pl.ds(c, LANES)] = jnp.right_shift(
                idx_v[r, pl.ds(c, LANES)], 1
            )

        @pl.loop(0, n_chunks)
        def chunk_loop(c):
            pltpu.async_copy(w_hbm.at[idxh_v.at[c]], rows_v, gsem).wait()

            @pl.loop(0, CHUNK // LANES)
            def group_loop(g):
                go = pl.multiple_of(g * LANES, LANES)
                mvec = jnp.bitwise_and(idx_v[c, pl.ds(go, LANES)], 1)
                for k in range(LANES):
                    off = pl.multiple_of(mvec[k] * D_MODEL, D_MODEL)
                    for j in range(D_MODEL // LANES):
                        rows_o[go + k, pl.ds(j * LANES, LANES)] = (
                            rows_v[go + k, pl.ds(off + j * LANES, LANES)] * SCALE
                        )

            pltpu.async_copy(
                rows_o, out_hbm.at[pl.ds(base + c * CHUNK, CHUNK)], osem
            ).wait()

    return body(w2, x3d)


def kernel(x, weight):
    b, t = x.shape
    n = b * t
    n_per_w = n // NUM_WORKERS
    n_chunks = n_per_w // CHUNK
    x3d = x.reshape(NUM_WORKERS, n_chunks, CHUNK).astype(jnp.int32)
    w2 = weight.reshape(weight.shape[0] // 2, D_PAD)
    outp = _embed_sc(x3d, w2)
    return outp.reshape(b, t, D_PAD)[:, :, :D_MODEL]


# speedup vs baseline: 1.2278x; 1.2278x over previous
import functools
import math

import jax
import jax.numpy as jnp
from jax import lax
from jax.experimental import pallas as pl
from jax.experimental.pallas import tpu as pltpu
from jax.experimental.pallas import tpu_sc as plsc

D_MODEL = 64
D_PAD = 128
SCALE = math.sqrt(D_MODEL)
LANES = 16

NUM_CORES = 2
NUM_SUBCORES = 16
NUM_WORKERS = NUM_CORES * NUM_SUBCORES

CHUNK = 128


@jax.jit
def _embed_sc(x3d, wpad):
    nw, n_chunks, _ = x3d.shape
    n_total = nw * n_chunks * CHUNK

    mesh = plsc.VectorSubcoreMesh(core_axis_name="c", subcore_axis_name="s")

    @functools.partial(
        pl.kernel,
        out_type=jax.ShapeDtypeStruct((n_total, D_PAD), jnp.float32),
        mesh=mesh,
        scratch_types=[
            pltpu.VMEM((n_chunks, CHUNK), jnp.int32),
            pltpu.VMEM((CHUNK, D_PAD), jnp.float32),
            pltpu.VMEM((CHUNK, D_PAD), jnp.float32),
            pltpu.SemaphoreType.DMA,
            pltpu.SemaphoreType.DMA,
        ],
        compiler_params=pltpu.CompilerParams(use_tc_tiling_on_sc=False),
    )
    def body(w_hbm, idx_hbm, out_hbm, idx_v, rows_v, rows_o, gsem, osem):
        wid = lax.axis_index("s") * NUM_CORES + lax.axis_index("c")
        base = wid * n_chunks * CHUNK
        pltpu.sync_copy(idx_hbm.at[wid], idx_v)

        @pl.loop(0, n_chunks)
        def chunk_loop(c):
            pltpu.async_copy(w_hbm.at[idx_v.at[c]], rows_v, gsem).wait()

            @pl.loop(0, CHUNK)
            def row_loop(r):
                for j in range(D_MODEL // LANES):
                    sl = pl.ds(j * LANES, LANES)
                    rows_o[r, sl] = rows_v[r, sl] * SCALE

            pltpu.async_copy(
                rows_o, out_hbm.at[pl.ds(base + c * CHUNK, CHUNK)], osem
            ).wait()

    return body(wpad, x3d)


def kernel(x, weight):
    b, t = x.shape
    n = b * t
    n_per_w = n // NUM_WORKERS
    n_chunks = n_per_w // CHUNK
    x3d = x.reshape(NUM_WORKERS, n_chunks, CHUNK).astype(jnp.int32)
    wpad = jnp.pad(weight, ((0, 0), (0, D_PAD - D_MODEL)))
    outp = _embed_sc(x3d, wpad)
    return outp.reshape(b, t, D_PAD)[:, :, :D_MODEL]
